# baseline (speedup 1.0000x reference)
import jax
import jax.numpy as jnp
from jax import lax
from jax.experimental import pallas as pl
from jax.experimental.pallas import tpu as pltpu
from jax.experimental.pallas import tpu_sc as plsc

C = 128
K = 32
NW = 32
LANES = 16
NBUF = 4
LOOKAHEAD = 3
ROWS_PER_GATHER = 64
NODES_PER_CHUNK = ROWS_PER_GATHER // K
CH_PER_GRP = 2
IDX_W = 128


def _tc_z_body(xn_ref, w1_ref, b1_ref, z_ref):
    n = xn_ref.shape[0]
    z = lax.dot_general(xn_ref[...], w1_ref[...], (((1,), (1,)), ((), ())),
                        preferred_element_type=jnp.float32)
    z_ref[0:n, :] = jnp.maximum(z + b1_ref[...], 0.0)


def _tc_p_body(xn_ref, w2a_ref, b2_ref, p_ref):
    p = lax.dot_general(xn_ref[...], w2a_ref[...], (((1,), (1,)), ((), ())),
                        preferred_element_type=jnp.float32)
    p_ref[...] = p + b2_ref[...]


def _tc_post_body(p_ref, m_ref, w2b_ref, out_ref):
    n = p_ref.shape[0]
    mm = lax.dot_general(m_ref[0:n, :], w2b_ref[...], (((1,), (1,)), ((), ())),
                         preferred_element_type=jnp.float32)
    out_ref[...] = jnp.maximum(p_ref[...] + mm, 0.0)


def _node_max(rows, ob, chunk_base, out_v):
    ng = C // LANES

    for p_node in range(NODES_PER_CHUNK):
        base = p_node * K
        acc = tuple(rows[base, pl.ds(g * LANES, LANES)] for g in range(ng))

        def body(r, a, _base=base):
            return tuple(
                jnp.maximum(a[g], rows[_base + r, pl.ds(g * LANES, LANES)])
                for g in range(ng))

        acc = lax.fori_loop(1, K, body, acc, unroll=4)
        for g in range(ng):
            out_v[ob, chunk_base + p_node, pl.ds(g * LANES, LANES)] = acc[g]


def _sc_gather_max(z_hbm, idx_hbm, m_hbm, idx_v, rows_v, out_v, z_sp,
                   gsem0, gsem1, gsem2, gsem3, wsem0, wsem1):
    nch = idx_hbm.shape[1] * 2
    npw = nch * NODES_PER_CHUNK
    npg = CH_PER_GRP * NODES_PER_CHUNK
    ngrp = nch // CH_PER_GRP
    gsems = (gsem0, gsem1, gsem2, gsem3)
    wsems = (wsem0, wsem1)
    sid = lax.axis_index("s")
    wid = lax.axis_index("c") * (NW // 2) + sid

    pltpu.sync_copy(idx_hbm.at[wid], idx_v)

    n_stage = z_hbm.shape[0] // (NW // 2)
    pltpu.sync_copy(z_hbm.at[pl.ds(sid * n_stage, n_stage)],
                    z_sp.at[pl.ds(sid * n_stage, n_stage)])
    plsc.subcore_barrier()

    def gather(cpair, half, b, from_hbm=False):
        idx_slice = idx_v.at[cpair, pl.ds(half * ROWS_PER_GATHER,
                                          ROWS_PER_GATHER)]
        src = z_hbm if from_hbm else z_sp
        return pltpu.make_async_copy(
            src.at[idx_slice], rows_v.at[b], gsems[b])

    def writeback(g, ob):
        return pltpu.make_async_copy(
            out_v.at[ob], m_hbm.at[pl.ds(wid * npw + g * npg, npg)],
            wsems[ob])

    for cp in range(LOOKAHEAD):
        gather(cp // 2, cp % 2, cp % NBUF).start()

    def supergroup(sg, carry):
        for ob in range(2):
            g = sg * 2 + ob

            @pl.when(g >= 2)
            def _():
                writeback(g - 2, ob).wait()

            for j in range(CH_PER_GRP):
                c = g * CH_PER_GRP + j
                b = (ob * CH_PER_GRP + j) % NBUF
                la = ob * CH_PER_GRP + j + LOOKAHEAD

                @pl.when(c + LOOKAHEAD < nch)
                def _():
                    gather(sg * 2 + la // 2, la % 2, la % NBUF,
                           la % 4 == 3).start()

                gather(g, j, b, (ob * CH_PER_GRP + j) % 4 == 3).wait()
                _node_max(rows_v.at[b], ob, j * NODES_PER_CHUNK, out_v)
            writeback(g, ob).start()
        return carry

    lax.fori_loop(0, ngrp // 2, supergroup, 0)
    writeback(ngrp - 2, 0).wait()
    writeback(ngrp - 1, 1).wait()


def kernel(x, edge_index, w1, b1, w2, b2):
    n = x.shape[2]
    xn = x[0, :, :, 0].T
    idx = edge_index[0, 0]

    chunks_pw = -(-n // (NW * NODES_PER_CHUNK))
    chunks_pw = -(-chunks_pw // (2 * CH_PER_GRP)) * (2 * CH_PER_GRP)
    npw = chunks_pw * NODES_PER_CHUNK
    n_pad = NW * npw
    idx_flat = jnp.pad(idx.reshape(-1), (0, (n_pad - n) * K))
    idx_grp = idx_flat.reshape(NW, chunks_pw // 2, IDX_W)

    z = pl.pallas_call(
        _tc_z_body,
        out_shape=jax.ShapeDtypeStruct((n_pad, C), jnp.float32),
    )(xn, w1, b1.reshape(1, C))

    p = pl.pallas_call(
        _tc_p_body,
        out_shape=jax.ShapeDtypeStruct((n, C), jnp.float32),
    )(xn, w2[:, :C], b2.reshape(1, C))

    sc_call = pl.kernel(
        _sc_gather_max,
        out_type=jax.ShapeDtypeStruct((n_pad, C), jnp.float32),
        mesh=plsc.VectorSubcoreMesh(core_axis_name="c", subcore_axis_name="s"),
        scratch_types=(
            pltpu.VMEM((chunks_pw // 2, IDX_W), jnp.int32),
            pltpu.VMEM((NBUF, ROWS_PER_GATHER, C), jnp.float32),
            pltpu.VMEM((2, CH_PER_GRP * NODES_PER_CHUNK, C), jnp.float32),
            pltpu.VMEM_SHARED((n_pad, C), jnp.float32),
        ) + (pltpu.SemaphoreType.DMA,) * 6,
    )
    m = sc_call(z, idx_grp)

    out = pl.pallas_call(
        _tc_post_body,
        out_shape=jax.ShapeDtypeStruct((n, C), jnp.float32),
    )(p, m, w2[:, C:])
    return out.T.reshape(1, C, n, 1)


# speedup vs baseline: 1.9517x; 1.9517x over previous
import jax
import jax.numpy as jnp
from jax import lax
from jax.experimental import pallas as pl
from jax.experimental.pallas import tpu as pltpu
from jax.experimental.pallas import tpu_sc as plsc

C = 128
K = 32
NW = 32
LANES = 16
NBUF = 4
LOOKAHEAD = 3
ROWS_PER_GATHER = 64
NODES_PER_CHUNK = ROWS_PER_GATHER // K
CH_PER_GRP = 2
IDX_W = 128


def _tc_z_body(xn_ref, w1_ref, b1_ref, z_ref):
    n = xn_ref.shape[0]
    z = lax.dot_general(xn_ref[...], w1_ref[...], (((1,), (1,)), ((), ())),
                        preferred_element_type=jnp.float32)
    z_ref[0:n, :] = jnp.maximum(z + b1_ref[...], 0.0)


def _tc_p_body(xn_ref, w2a_ref, b2_ref, p_ref):
    p = lax.dot_general(xn_ref[...], w2a_ref[...], (((1,), (1,)), ((), ())),
                        preferred_element_type=jnp.float32)
    p_ref[...] = p + b2_ref[...]


def _tc_post_body(p_ref, m_ref, w2b_ref, out_ref):
    n = p_ref.shape[0]
    mm = lax.dot_general(m_ref[0:n, :], w2b_ref[...], (((1,), (1,)), ((), ())),
                         preferred_element_type=jnp.float32)
    out_ref[...] = jnp.maximum(p_ref[...] + mm, 0.0)


def _node_max(rows, ob, chunk_base, out_v):
    ng = C // LANES

    for p_node in range(NODES_PER_CHUNK):
        base = p_node * K
        acc = tuple(rows[base, pl.ds(g * LANES, LANES)] for g in range(ng))

        def body(r, a, _base=base):
            return tuple(
                jnp.maximum(a[g], rows[_base + r, pl.ds(g * LANES, LANES)])
                for g in range(ng))

        acc = lax.fori_loop(1, K, body, acc, unroll=4)
        for g in range(ng):
            out_v[ob, chunk_base + p_node, pl.ds(g * LANES, LANES)] = acc[g]


def _sc_gather_max(z_hbm, idx_hbm, m_hbm, idx_v, rows_v, out_v, z_sp,
                   gsem0, gsem1, gsem2, gsem3, wsem0, wsem1):
    nch = idx_hbm.shape[1] * 2
    npw = nch * NODES_PER_CHUNK
    npg = CH_PER_GRP * NODES_PER_CHUNK
    ngrp = nch // CH_PER_GRP
    gsems = (gsem0, gsem1, gsem2, gsem3)
    wsems = (wsem0, wsem1)
    sid = lax.axis_index("s")
    wid = lax.axis_index("c") * (NW // 2) + sid

    pltpu.sync_copy(idx_hbm.at[wid], idx_v)

    n_stage = z_hbm.shape[0] // (NW // 2)
    pltpu.sync_copy(z_hbm.at[pl.ds(sid * n_stage, n_stage)],
                    z_sp.at[pl.ds(sid * n_stage, n_stage)])
    plsc.subcore_barrier()

    def gather(cpair, half, b):
        idx_slice = idx_v.at[cpair, pl.ds(half * ROWS_PER_GATHER,
                                          ROWS_PER_GATHER)]
        return pltpu.make_async_copy(
            z_sp.at[idx_slice], rows_v.at[b], gsems[b])

    def writeback(g, ob):
        return pltpu.make_async_copy(
            out_v.at[ob], m_hbm.at[pl.ds(wid * npw + g * npg, npg)],
            wsems[ob])

    for cp in range(LOOKAHEAD):
        gather(cp // 2, cp % 2, cp % NBUF).start()

    def supergroup(sg, carry):
        for ob in range(2):
            g = sg * 2 + ob

            @pl.when(g >= 2)
            def _():
                writeback(g - 2, ob).wait()

            for j in range(CH_PER_GRP):
                c = g * CH_PER_GRP + j
                b = (ob * CH_PER_GRP + j) % NBUF
                la = ob * CH_PER_GRP + j + LOOKAHEAD

                @pl.when(c + LOOKAHEAD < nch)
                def _():
                    gather(sg * 2 + la // 2, la % 2, la % NBUF).start()

                gather(g, j, b).wait()
                _node_max(rows_v.at[b], ob, j * NODES_PER_CHUNK, out_v)
            writeback(g, ob).start()
        return carry

    lax.fori_loop(0, ngrp // 2, supergroup, 0)
    writeback(ngrp - 2, 0).wait()
    writeback(ngrp - 1, 1).wait()


def kernel(x, edge_index, w1, b1, w2, b2):
    n = x.shape[2]
    xn = x[0, :, :, 0].T
    idx = edge_index[0, 0]

    chunks_pw = -(-n // (NW * NODES_PER_CHUNK))
    chunks_pw = -(-chunks_pw // (2 * CH_PER_GRP)) * (2 * CH_PER_GRP)
    npw = chunks_pw * NODES_PER_CHUNK
    n_pad = NW * npw
    idx_flat = jnp.pad(idx.reshape(-1), (0, (n_pad - n) * K))
    idx_grp = idx_flat.reshape(NW, chunks_pw // 2, IDX_W)

    z = pl.pallas_call(
        _tc_z_body,
        out_shape=jax.ShapeDtypeStruct((n_pad, C), jnp.float32),
    )(xn, w1, b1.reshape(1, C))

    p = pl.pallas_call(
        _tc_p_body,
        out_shape=jax.ShapeDtypeStruct((n, C), jnp.float32),
    )(xn, w2[:, :C], b2.reshape(1, C))

    sc_call = pl.kernel(
        _sc_gather_max,
        out_type=jax.ShapeDtypeStruct((n_pad, C), jnp.float32),
        mesh=plsc.VectorSubcoreMesh(core_axis_name="c", subcore_axis_name="s"),
        scratch_types=(
            pltpu.VMEM((chunks_pw // 2, IDX_W), jnp.int32),
            pltpu.VMEM((NBUF, ROWS_PER_GATHER, C), jnp.float32),
            pltpu.VMEM((2, CH_PER_GRP * NODES_PER_CHUNK, C), jnp.float32),
            pltpu.VMEM_SHARED((n_pad, C), jnp.float32),
        ) + (pltpu.SemaphoreType.DMA,) * 6,
    )
    m = sc_call(z, idx_grp)

    out = pl.pallas_call(
        _tc_post_body,
        out_shape=jax.ShapeDtypeStruct((n, C), jnp.float32),
    )(p, m, w2[:, C:])
    return out.T.reshape(1, C, n, 1)
